# Initial kernel scaffold; baseline (speedup 1.0000x reference)
import functools

import jax
import jax.numpy as jnp
from jax import lax
from jax.experimental import pallas as pl
from jax.experimental.pallas import tpu as pltpu
from jax.experimental.pallas import tpu_sc as plsc

N = 10000
E = 320000
D = 128

NC = 2
NS = 16
NW = NC * NS
EPW = E // NW
CH = 80
STEPS = EPW // CH
RPS = N // NS

_mesh = plsc.VectorSubcoreMesh(core_axis_name="c", subcore_axis_name="s")


@functools.partial(
    pl.kernel,
    out_type=jax.ShapeDtypeStruct((NC * N, D), jnp.float32),
    mesh=_mesh,
    scratch_types=[
        pltpu.VMEM((CH,), jnp.int32),
        pltpu.VMEM((CH,), jnp.int32),
        pltpu.VMEM((CH, D), jnp.float32),
        pltpu.VMEM_SHARED((N, D), jnp.float32),
        pltpu.SemaphoreType.DMA,
    ],
)
def _segsum(h_hbm, src_hbm, dst_hbm, z_hbm, out_hbm, sidx, didx, rows, acc, sem):
    cid = lax.axis_index("c")
    sid = lax.axis_index("s")
    wid = cid * NS + sid

    zbase = sid * RPS
    pltpu.sync_copy(z_hbm.at[pl.ds(zbase, RPS)], acc.at[pl.ds(zbase, RPS)])
    plsc.subcore_barrier()

    ebase = wid * EPW

    def step(i, _):
        base = pl.multiple_of(ebase + i * CH, 8)
        pltpu.sync_copy(src_hbm.at[pl.ds(base, CH)], sidx)
        pltpu.sync_copy(dst_hbm.at[pl.ds(base, CH)], didx)
        pltpu.async_copy(h_hbm.at[sidx], rows, sem).wait()
        pltpu.sync_copy(rows, acc.at[didx], add=True)
        return 0

    lax.fori_loop(0, STEPS, step, 0)
    plsc.subcore_barrier()

    obase = cid * N + sid * RPS
    pltpu.sync_copy(acc.at[pl.ds(sid * RPS, RPS)], out_hbm.at[pl.ds(obase, RPS)])


def _update_body(relu, m_ref, norm_ref, w_ref, b_ref, o_ref):
    m = m_ref[0] + m_ref[1]
    m = m * norm_ref[...]
    acc = jnp.dot(m, w_ref[...], preferred_element_type=jnp.float32)
    acc = acc + b_ref[...]
    if relu:
        acc = jnp.maximum(acc, 0.0)
    o_ref[...] = acc


def _update(m2, norm, W, b, relu):
    BN = 1000
    H = W.shape[1]
    return pl.pallas_call(
        functools.partial(_update_body, relu),
        grid=(N // BN,),
        in_specs=[
            pl.BlockSpec((2, BN, D), lambda i: (0, i, 0)),
            pl.BlockSpec((BN, 1), lambda i: (i, 0)),
            pl.BlockSpec((D, H), lambda i: (0, 0)),
            pl.BlockSpec((1, H), lambda i: (0, 0)),
        ],
        out_specs=pl.BlockSpec((BN, H), lambda i: (i, 0)),
        out_shape=jax.ShapeDtypeStruct((N, H), jnp.float32),
    )(m2, norm, W, b.reshape(1, H))


@jax.jit
def kernel(x, edge_index, norm, W0, b0, W1, b1, W2, b2):
    src = edge_index[0]
    dst = edge_index[1]
    z = jnp.zeros((N, D), jnp.float32)

    m = _segsum(x, src, dst, z).reshape(2, N, D)
    h = _update(m, norm, W0, b0, relu=True)
    m = _segsum(h, src, dst, z).reshape(2, N, D)
    h = _update(m, norm, W1, b1, relu=True)
    m = _segsum(h, src, dst, z).reshape(2, N, D)
    h = _update(m, norm, W2, b2, relu=False)
    return h


# speedup vs baseline: 4.5475x; 4.5475x over previous
import functools

import jax
import jax.numpy as jnp
from jax import lax
from jax.experimental import pallas as pl
from jax.experimental.pallas import tpu as pltpu
from jax.experimental.pallas import tpu_sc as plsc

N = 10000
E = 320000
D = 128

NC = 2
NS = 16
NW = NC * NS
EPW = E // NW
CH = 80
STEPS = EPW // CH
NPAD = 10240
RPS = NPAD // NS

_mesh = plsc.VectorSubcoreMesh(core_axis_name="c", subcore_axis_name="s")


@functools.partial(
    pl.kernel,
    out_type=jax.ShapeDtypeStruct((NC * NPAD, D), jnp.float32),
    mesh=_mesh,
    scratch_types=[
        pltpu.VMEM((CH,), jnp.int32),
        pltpu.VMEM((CH,), jnp.int32),
        pltpu.VMEM((CH, D), jnp.float32),
        pltpu.VMEM_SHARED((NPAD, D), jnp.float32),
        pltpu.SemaphoreType.DMA,
    ],
)
def _segsum(h_hbm, src_hbm, dst_hbm, z_hbm, out_hbm, sidx, didx, rows, acc, sem):
    cid = lax.axis_index("c")
    sid = lax.axis_index("s")
    wid = cid * NS + sid

    zbase = sid * RPS
    pltpu.sync_copy(z_hbm.at[pl.ds(zbase, RPS)], acc.at[pl.ds(zbase, RPS)])
    plsc.subcore_barrier()

    ebase = wid * EPW

    def step(i, _):
        base = pl.multiple_of(ebase + i * CH, 8)
        pltpu.sync_copy(src_hbm.at[pl.ds(base, CH)], sidx)
        pltpu.sync_copy(dst_hbm.at[pl.ds(base, CH)], didx)
        pltpu.async_copy(h_hbm.at[sidx], rows, sem).wait()
        pltpu.sync_copy(rows, acc.at[didx], add=True)
        return 0

    lax.fori_loop(0, STEPS, step, 0)
    plsc.subcore_barrier()

    obase = cid * NPAD + sid * RPS
    pltpu.sync_copy(acc.at[pl.ds(sid * RPS, RPS)], out_hbm.at[pl.ds(obase, RPS)])


def _update_body(relu, m_ref, norm_ref, w_ref, b_ref, o_ref):
    m = m_ref[0] + m_ref[1]
    m = m * norm_ref[...]
    acc = jnp.dot(m, w_ref[...], preferred_element_type=jnp.float32)
    acc = acc + b_ref[...]
    if relu:
        acc = jnp.maximum(acc, 0.0)
    o_ref[...] = acc


def _update(m2, norm, W, b, relu):
    BN = 1000
    H = W.shape[1]
    return pl.pallas_call(
        functools.partial(_update_body, relu),
        grid=(N // BN,),
        in_specs=[
            pl.BlockSpec((2, BN, D), lambda i: (0, i, 0)),
            pl.BlockSpec((BN, 1), lambda i: (i, 0)),
            pl.BlockSpec((D, H), lambda i: (0, 0)),
            pl.BlockSpec((1, H), lambda i: (0, 0)),
        ],
        out_specs=pl.BlockSpec((BN, H), lambda i: (i, 0)),
        out_shape=jax.ShapeDtypeStruct((N, H), jnp.float32),
    )(m2, norm, W, b.reshape(1, H))


@jax.jit
def kernel(x, edge_index, norm, W0, b0, W1, b1, W2, b2):
    src = edge_index[0]
    dst = edge_index[1]
    z = jnp.zeros((NPAD, D), jnp.float32)

    m = _segsum(x, src, dst, z).reshape(2, NPAD, D)
    h = _update(m, norm, W0, b0, relu=True)
    m = _segsum(h, src, dst, z).reshape(2, NPAD, D)
    h = _update(m, norm, W1, b1, relu=True)
    m = _segsum(h, src, dst, z).reshape(2, NPAD, D)
    h = _update(m, norm, W2, b2, relu=False)
    return h
